# baseline (speedup 1.0000x reference)
from functools import partial

import jax
import jax.numpy as jnp
import numpy as np
from jax.experimental import pallas as pl
from jax.experimental.pallas import tpu as pltpu
from jax.experimental.shard_map import shard_map
from jax.sharding import Mesh, PartitionSpec as P

BLOCK_B = 32


def _linear_rows_kernel(x_ref, wt_ref, b_ref, o_ref):
    acc = jax.lax.dot_general(
        x_ref[...], wt_ref[...],
        dimension_numbers=(((1,), (0,)), ((), ())),
        preferred_element_type=jnp.float32,
    )
    o_ref[...] = acc + b_ref[...]


def _local_linear(x, wt, b2):
    batch, k = x.shape
    n_local = wt.shape[1]
    grid = (batch // BLOCK_B,)
    return pl.pallas_call(
        _linear_rows_kernel,
        grid=grid,
        in_specs=[
            pl.BlockSpec((BLOCK_B, k), lambda i: (i, 0)),
            pl.BlockSpec((k, n_local), lambda i: (0, 0)),
            pl.BlockSpec((1, n_local), lambda i: (0, 0)),
        ],
        out_specs=pl.BlockSpec((BLOCK_B, n_local), lambda i: (i, 0)),
        out_shape=jax.ShapeDtypeStruct((batch, n_local), jnp.float32),
        compiler_params=pltpu.CompilerParams(
            dimension_semantics=("parallel",),
        ),
    )(x, wt, b2)


@jax.jit
def kernel(x, W, b):
    num_classes = W.shape[0]
    wt = W.T
    b2 = b.reshape(1, num_classes)
    devs = jax.devices()
    n_dev = 2 if (len(devs) >= 2 and num_classes % 2 == 0) else 1
    if n_dev == 1:
        return _local_linear(x, wt, b2)
    mesh = Mesh(np.array(devs[:n_dev]), ("c",))
    f = shard_map(
        _local_linear,
        mesh=mesh,
        in_specs=(P(None, None), P(None, "c"), P(None, "c")),
        out_specs=P(None, "c"),
        check_rep=False,
    )
    return f(x, wt, b2)


# speedup vs baseline: 4.1772x; 4.1772x over previous
import jax
import jax.numpy as jnp
from jax.experimental import pallas as pl
from jax.experimental.pallas import tpu as pltpu

BLOCK_C = 4096


def _linear_t_kernel(wbt_ref, xt_ref, o_ref):
    o_ref[...] = jax.lax.dot_general(
        wbt_ref[...], xt_ref[...],
        dimension_numbers=(((0,), (0,)), ((), ())),
        preferred_element_type=jnp.float32,
    )


@jax.jit
def kernel(x, W, b):
    batch, k = x.shape
    num_classes = W.shape[0]
    wbt = jnp.concatenate([W.T, b.reshape(1, num_classes)], axis=0)
    xt1 = jnp.concatenate(
        [x, jnp.ones((batch, 1), jnp.float32)], axis=1
    ).T
    grid = (pl.cdiv(num_classes, BLOCK_C),)
    out_t = pl.pallas_call(
        _linear_t_kernel,
        grid=grid,
        in_specs=[
            pl.BlockSpec((k + 1, BLOCK_C), lambda i: (0, i)),
            pl.BlockSpec((k + 1, batch), lambda i: (0, 0)),
        ],
        out_specs=pl.BlockSpec((BLOCK_C, batch), lambda i: (i, 0)),
        out_shape=jax.ShapeDtypeStruct((num_classes, batch), jnp.float32),
        compiler_params=pltpu.CompilerParams(
            dimension_semantics=("parallel",),
        ),
    )(wbt, xt1)
    return out_t.T


# speedup vs baseline: 4.6054x; 1.1025x over previous
import jax
import jax.numpy as jnp
from jax.experimental import pallas as pl
from jax.experimental.pallas import tpu as pltpu

BLOCK_C = 4096


def _linear_t_kernel(wbt_ref, xt_ref, o_ref):
    o_ref[...] = jax.lax.dot_general(
        wbt_ref[...], xt_ref[...],
        dimension_numbers=(((0,), (0,)), ((), ())),
        preferred_element_type=jnp.float32,
    )


@jax.jit
def kernel(x, W, b):
    batch, k = x.shape
    num_classes = W.shape[0]
    wbt = jnp.concatenate([W.T, b.reshape(1, num_classes)], axis=0)
    xt1 = jnp.concatenate(
        [x, jnp.ones((batch, 1), jnp.float32)], axis=1
    ).T
    grid = (pl.cdiv(num_classes, BLOCK_C),)
    out_t = pl.pallas_call(
        _linear_t_kernel,
        grid=grid,
        in_specs=[
            pl.BlockSpec((k + 1, BLOCK_C), lambda i: (0, i)),
            pl.BlockSpec((k + 1, batch), lambda i: (0, 0)),
        ],
        out_specs=pl.BlockSpec((BLOCK_C, batch), lambda i: (i, 0)),
        out_shape=jax.ShapeDtypeStruct((num_classes, batch), jnp.float32),
        compiler_params=pltpu.CompilerParams(
            dimension_semantics=("parallel",),
            allow_input_fusion=(True, True),
        ),
    )(wbt, xt1)
    return out_t.T
